# Initial kernel scaffold; baseline (speedup 1.0000x reference)
import jax
import jax.numpy as jnp
from jax.experimental import pallas as pl


def kernel(tokens, features, emb, lw_ih_f, lw_hh_f, lb_ih_f, lb_hh_f, lw_ih_r, lw_hh_r, lb_ih_r, lb_hh_r, gw_ih_f, gw_hh_f, gb_ih_f, gb_hh_f, gw_ih_r, gw_hh_r, gb_ih_r, gb_hh_r, W1, b1, W2, b2):
    raise NotImplementedError("write your pallas kernel here")


# speedup vs baseline: 2.3586x; 2.3586x over previous
import functools

import jax
import jax.numpy as jnp
from jax import lax
from jax.experimental import pallas as pl
from jax.experimental.pallas import tpu as pltpu
from jax.experimental.pallas import tpu_sc as plsc

H = 64
E = 300
B = 1024
L = 200
NF = 4
LIN = 128


V = 120000
PROJ = 8 * H
_MBLK = 3000


def _proj_body(e_ref, w_ref, o_ref):
    o_ref[...] = jnp.dot(e_ref[...], w_ref[...],
                         preferred_element_type=jnp.float32)


def _proj_table(emb, wcat):
    return pl.pallas_call(
        _proj_body,
        grid=(V // _MBLK,),
        in_specs=[
            pl.BlockSpec((_MBLK, E), lambda i: (i, 0)),
            pl.BlockSpec((E, PROJ), lambda i: (0, 0)),
        ],
        out_specs=pl.BlockSpec((_MBLK, PROJ), lambda i: (i, 0)),
        out_shape=jax.ShapeDtypeStruct((V, PROJ), jnp.float32),
        compiler_params=pltpu.CompilerParams(
            dimension_semantics=("arbitrary",)),
    )(emb, wcat)


_NC = 2
_NS = 16
_NW = _NC * _NS
_N_TOK = L * B
_PER_W = _N_TOK // _NW
_CHUNK = 64
_NCHUNK = _PER_W // _CHUNK


def _sc_gather(tok_flat, table):
    mesh = plsc.VectorSubcoreMesh(core_axis_name="c", subcore_axis_name="s")

    @functools.partial(
        pl.kernel,
        mesh=mesh,
        out_type=jax.ShapeDtypeStruct((_N_TOK, PROJ), jnp.float32),
        scratch_types=[
            pltpu.VMEM((_PER_W,), jnp.int32),
            pltpu.VMEM((_CHUNK, PROJ), jnp.float32),
            pltpu.VMEM((_CHUNK, PROJ), jnp.float32),
            pltpu.SemaphoreType.DMA,
            pltpu.SemaphoreType.DMA,
        ],
    )
    def k(tok_hbm, tbl_hbm, out_hbm, idx_v, buf0, buf1, sem0, sem1):
        wid = lax.axis_index("s") * _NC + lax.axis_index("c")
        base = wid * _PER_W
        pltpu.sync_copy(tok_hbm.at[pl.ds(base, _PER_W)], idx_v)

        bufs = (buf0, buf1)
        sems = (sem0, sem1)

        def body(i, _):
            for j in range(2):
                ci = i * 2 + j
                g = pltpu.async_copy(
                    tbl_hbm.at[idx_v.at[pl.ds(ci * _CHUNK, _CHUNK)]],
                    bufs[j], sems[j])
                g.wait()
                pltpu.sync_copy(bufs[j],
                                out_hbm.at[pl.ds(base + ci * _CHUNK, _CHUNK)])
            return 0

        lax.fori_loop(0, _NCHUNK // 2, body, 0)

    return k(tok_flat, table)


def _bilstm_body(xf_ref, xr_ref, uf_ref, bf_ref, ur_ref,
                 br_ref, hf_out, hr_out, hf_s, cf_s, hr_s, cr_s):
    t = pl.program_id(0)

    @pl.when(t == 0)
    def _():
        hf_s[...] = jnp.zeros_like(hf_s)
        cf_s[...] = jnp.zeros_like(cf_s)
        hr_s[...] = jnp.zeros_like(hr_s)
        cr_s[...] = jnp.zeros_like(cr_s)

    def step(xp, u_ref, b_ref, h_s, c_s, h_out):
        h_prev = h_s[...]
        c_prev = c_s[...]
        g = (xp
             + jnp.dot(h_prev, u_ref[...],
                       preferred_element_type=jnp.float32)
             + b_ref[...])
        i = jax.nn.sigmoid(g[:, 0:H])
        f = jax.nn.sigmoid(g[:, H:2 * H])
        gg = jnp.tanh(g[:, 2 * H:3 * H])
        o = jax.nn.sigmoid(g[:, 3 * H:4 * H])
        c = f * c_prev + i * gg
        h = o * jnp.tanh(c)
        c_s[...] = c
        h_s[...] = h
        h_out[0] = h

    step(xf_ref[0], uf_ref, bf_ref, hf_s, cf_s, hf_out)
    step(xr_ref[0], ur_ref, br_ref, hr_s, cr_s, hr_out)


def _bilstm(xp, uf, bf, ur, br):
    return pl.pallas_call(
        _bilstm_body,
        grid=(L,),
        in_specs=[
            pl.BlockSpec((1, B, 4 * H), lambda t: (t, 0, 0)),
            pl.BlockSpec((1, B, 4 * H), lambda t: (L - 1 - t, 0, 1)),
            pl.BlockSpec((H, 4 * H), lambda t: (0, 0)),
            pl.BlockSpec((1, 4 * H), lambda t: (0, 0)),
            pl.BlockSpec((H, 4 * H), lambda t: (0, 0)),
            pl.BlockSpec((1, 4 * H), lambda t: (0, 0)),
        ],
        out_specs=[
            pl.BlockSpec((1, B, H), lambda t: (t, 0, 0)),
            pl.BlockSpec((1, B, H), lambda t: (t, 0, 0)),
        ],
        out_shape=[
            jax.ShapeDtypeStruct((L, B, H), jnp.float32),
            jax.ShapeDtypeStruct((L, B, H), jnp.float32),
        ],
        scratch_shapes=[pltpu.VMEM((B, H), jnp.float32)] * 4,
        compiler_params=pltpu.CompilerParams(
            dimension_semantics=("arbitrary",)),
    )(xp, xp, uf, bf, ur, br)


def _bigru_body(hf_f_ref, hr_f_ref, hf_r_ref, hr_r_ref,
                wgf_ref, ugf_ref, bif_ref, bhf_ref,
                wgr_ref, ugr_ref, bir_ref, bhr_ref,
                hTf_o, hTr_o, avg_f_o, avg_r_o, max_f_o, max_r_o,
                hgf_s, hgr_s):
    t = pl.program_id(0)
    first = t == 0

    def step(hl, w_ref, u_ref, bi_ref, bh_ref, h_s, avg_o, max_o):
        h_prev = jnp.where(first, 0.0, h_s[...])
        gi = (jnp.dot(hl, w_ref[...], preferred_element_type=jnp.float32)
              + bi_ref[...])
        gh = (jnp.dot(h_prev, u_ref[...],
                      preferred_element_type=jnp.float32)
              + bh_ref[...])
        r = jax.nn.sigmoid(gi[:, 0:H] + gh[:, 0:H])
        z = jax.nn.sigmoid(gi[:, H:2 * H] + gh[:, H:2 * H])
        n = jnp.tanh(gi[:, 2 * H:3 * H] + r * gh[:, 2 * H:3 * H])
        h = (1.0 - z) * n + z * h_prev
        h_s[...] = h
        avg_o[...] = jnp.where(first, 0.0, avg_o[...]) + h
        max_o[...] = jnp.maximum(
            jnp.where(first, -jnp.inf, max_o[...]), h)

    hl_f = jnp.concatenate([hf_f_ref[0], hr_f_ref[0]], axis=1)
    step(hl_f, wgf_ref, ugf_ref, bif_ref, bhf_ref, hgf_s, avg_f_o, max_f_o)
    hl_r = jnp.concatenate([hf_r_ref[0], hr_r_ref[0]], axis=1)
    step(hl_r, wgr_ref, ugr_ref, bir_ref, bhr_ref, hgr_s, avg_r_o, max_r_o)

    @pl.when(t == L - 1)
    def _():
        hTf_o[...] = hgf_s[...]
        hTr_o[...] = hgr_s[...]
        avg_f_o[...] = avg_f_o[...] * (1.0 / L)
        avg_r_o[...] = avg_r_o[...] * (1.0 / L)


def _bigru(hf, hr, wgf, ugf, bif, bhf, wgr, ugr, bir, bhr):
    const2 = lambda t: (0, 0)
    seq = lambda t: (t, 0, 0)
    rev = lambda t: (L - 1 - t, 0, 0)
    bh = pl.BlockSpec((B, H), const2)
    return pl.pallas_call(
        _bigru_body,
        grid=(L,),
        in_specs=[
            pl.BlockSpec((1, B, H), seq),
            pl.BlockSpec((1, B, H), rev),
            pl.BlockSpec((1, B, H), rev),
            pl.BlockSpec((1, B, H), seq),
            pl.BlockSpec((2 * H, 3 * H), const2),
            pl.BlockSpec((H, 3 * H), const2),
            pl.BlockSpec((1, 3 * H), const2),
            pl.BlockSpec((1, 3 * H), const2),
            pl.BlockSpec((2 * H, 3 * H), const2),
            pl.BlockSpec((H, 3 * H), const2),
            pl.BlockSpec((1, 3 * H), const2),
            pl.BlockSpec((1, 3 * H), const2),
        ],
        out_specs=[bh] * 6,
        out_shape=[jax.ShapeDtypeStruct((B, H), jnp.float32)] * 6,
        scratch_shapes=[pltpu.VMEM((B, H), jnp.float32)] * 2,
        compiler_params=pltpu.CompilerParams(
            dimension_semantics=("arbitrary",)),
    )(hf, hr, hf, hr, wgf, ugf, bif, bhf, wgr, ugr, bir, bhr)


def _head_body(conc_ref, w1_ref, b1_ref, w2_ref, b2_ref, out_ref):
    hid = jnp.dot(conc_ref[...], w1_ref[...],
                  preferred_element_type=jnp.float32) + b1_ref[...]
    hid = jnp.maximum(hid, 0.0)
    out_ref[...] = jnp.dot(hid, w2_ref[...],
                           preferred_element_type=jnp.float32) + b2_ref[...]


def _head(conc, w1t, b1, w2t, b2):
    return pl.pallas_call(
        _head_body,
        out_shape=jax.ShapeDtypeStruct((B, 1), jnp.float32),
    )(conc, w1t, b1.reshape(1, LIN), w2t, b2.reshape(1, 1))


def kernel(tokens, features, emb, lw_ih_f, lw_hh_f, lb_ih_f, lb_hh_f,
           lw_ih_r, lw_hh_r, lb_ih_r, lb_hh_r, gw_ih_f, gw_hh_f, gb_ih_f,
           gb_hh_f, gw_ih_r, gw_hh_r, gb_ih_r, gb_hh_r, W1, b1, W2, b2):
    tok_flat = tokens.T.reshape(_N_TOK).astype(jnp.int32)
    wcat = jnp.concatenate([lw_ih_f.T, lw_ih_r.T], axis=1)
    table = _proj_table(emb, wcat)
    xp = _sc_gather(tok_flat, table).reshape(L, B, PROJ)

    hf, hr = _bilstm(
        xp,
        lw_hh_f.T, (lb_ih_f + lb_hh_f).reshape(1, 4 * H),
        lw_hh_r.T, (lb_ih_r + lb_hh_r).reshape(1, 4 * H),
    )

    hTf, hTr, avg_f, avg_r, max_f, max_r = _bigru(
        hf, hr,
        gw_ih_f.T, gw_hh_f.T, gb_ih_f.reshape(1, 3 * H),
        gb_hh_f.reshape(1, 3 * H),
        gw_ih_r.T, gw_hh_r.T, gb_ih_r.reshape(1, 3 * H),
        gb_hh_r.reshape(1, 3 * H),
    )

    hh = jnp.stack([hTf, hTr], axis=0).reshape(B, 2 * H)
    conc = jnp.concatenate(
        [hh, avg_f, avg_r, max_f, max_r, features], axis=1)
    return _head(conc, W1.T, b1, W2.T, b2)
